# baseline (speedup 1.0000x reference)
import functools

import jax
import jax.numpy as jnp
from jax import lax
from jax.experimental import pallas as pl
from jax.experimental.pallas import tpu as pltpu
from jax.experimental.pallas import tpu_sc as plsc

_BUCKETS = 320
_EMBED = 64
_NC = 2
_NS = 16
_NW = _NC * _NS
_LANES = 16


def _qr_body(total, per_w, chunk, idx_hbm, qt_hbm, rt_hbm, out_hbm3,
             qt_v, rt_v, idx_v, buf0, buf1, sem0, sem1):
  nchunk = per_w // chunk
  rows_per_chunk = chunk // 50
  wid = lax.axis_index("s") * _NC + lax.axis_index("c")
  base = wid * per_w

  pltpu.sync_copy(qt_hbm, qt_v)
  pltpu.sync_copy(rt_hbm, rt_v)
  pltpu.sync_copy(idx_hbm.at[pl.ds(base, per_w)], idx_v)

  def compute(c, buf):
    @plsc.parallel_loop(0, chunk // _LANES, unroll=5)
    def group(g):
      iv = idx_v[pl.ds(c * chunk + g * _LANES, _LANES)]
      q = lax.shift_right_logical(lax.shift_right_logical(iv, 6) * 6554, 15)
      r = iv - q * _BUCKETS
      qb = q * _EMBED
      rb = r * _EMBED
      for l in range(_LANES):
        qbl = qb[l]
        rbl = rb[l]
        i = g * _LANES + l
        b = lax.shift_right_logical(i * 82, 12)
        h = i - b * 50
        for t in range(_EMBED // _LANES):
          qv = qt_v[pl.ds(qbl + t * _LANES, _LANES)]
          rv = rt_v[pl.ds(rbl + t * _LANES, _LANES)]
          buf[b, h, pl.ds(t * _LANES, _LANES)] = qv * rv

  row_base = base // 50

  def start_copy(c, buf, sem):
    cp = pltpu.make_async_copy(
        buf, out_hbm3.at[pl.ds(row_base + c * rows_per_chunk, rows_per_chunk)],
        sem)
    cp.start()

  def drain(buf, sem):
    pltpu.make_async_copy(
        buf, out_hbm3.at[pl.ds(row_base, rows_per_chunk)], sem).wait()

  compute(0, buf0)
  start_copy(0, buf0, sem0)
  compute(1, buf1)
  start_copy(1, buf1, sem1)

  def pair(p, carry):
    c0 = 2 * p
    drain(buf0, sem0)
    compute(c0, buf0)
    start_copy(c0, buf0, sem0)
    drain(buf1, sem1)
    compute(c0 + 1, buf1)
    start_copy(c0 + 1, buf1, sem1)
    return carry

  lax.fori_loop(1, nchunk // 2, pair, 0)
  drain(buf0, sem0)
  drain(buf1, sem1)


def kernel(inputs, q_table, r_table):
  total = inputs.shape[0] * inputs.shape[1]
  per_w = total // _NW
  chunk = 400
  idx = jnp.reshape(inputs, (total,)).astype(jnp.int32)
  qt = jnp.reshape(q_table, (_BUCKETS * _EMBED,))
  rt = jnp.reshape(r_table, (_BUCKETS * _EMBED,))

  mesh = plsc.VectorSubcoreMesh(core_axis_name="c", subcore_axis_name="s")
  body = functools.partial(_qr_body, total, per_w, chunk)
  out = pl.kernel(
      body,
      out_type=jax.ShapeDtypeStruct(
          (inputs.shape[0], inputs.shape[1], _EMBED), jnp.float32),
      mesh=mesh,
      compiler_params=pltpu.CompilerParams(
          needs_layout_passes=False, use_tc_tiling_on_sc=False),
      scratch_types=[
          pltpu.VMEM((_BUCKETS * _EMBED,), jnp.float32),
          pltpu.VMEM((_BUCKETS * _EMBED,), jnp.float32),
          pltpu.VMEM((per_w,), jnp.int32),
          pltpu.VMEM((chunk // 50, 50, _EMBED), jnp.float32),
          pltpu.VMEM((chunk // 50, 50, _EMBED), jnp.float32),
          pltpu.SemaphoreType.DMA,
          pltpu.SemaphoreType.DMA,
      ],
  )(idx, qt, rt)
  return out


# speedup vs baseline: 1.4237x; 1.4237x over previous
import functools

import jax
import jax.numpy as jnp
from jax import lax
from jax.experimental import pallas as pl
from jax.experimental.pallas import tpu as pltpu
from jax.experimental.pallas import tpu_sc as plsc

_BUCKETS = 320
_EMBED = 64
_NC = 2
_NS = 16
_NW = _NC * _NS
_LANES = 16


def _qr_body(total, per_w, chunk, idx_hbm, qt_hbm, rt_hbm, out_hbm3,
             qt_v, rt_v, idx_v, buf0, buf1, sem0, sem1):
  nchunk = per_w // chunk
  rows_per_chunk = chunk // 50
  wid = lax.axis_index("s") * _NC + lax.axis_index("c")
  base = wid * per_w

  pltpu.sync_copy(qt_hbm, qt_v)
  pltpu.sync_copy(rt_hbm, rt_v)
  pltpu.sync_copy(idx_hbm.at[pl.ds(base, per_w)], idx_v)

  def compute(c, buf):
    @plsc.parallel_loop(0, chunk // _LANES, unroll=2)
    def group(g):
      iv = idx_v[pl.ds(c * chunk + g * _LANES, _LANES)]
      q = lax.shift_right_logical(lax.shift_right_logical(iv, 6) * 6554, 15)
      r = iv - q * _BUCKETS
      qb = q * _EMBED
      rb = r * _EMBED
      for l in range(_LANES):
        qbl = qb[l]
        rbl = rb[l]
        i = g * _LANES + l
        b = lax.shift_right_logical(i * 82, 12)
        h = i - b * 50
        for u in range(_EMBED // 32):
          qv = qt_v[pl.ds(qbl + u * 32, 32)]
          rv = rt_v[pl.ds(rbl + u * 32, 32)]
          p = qv * rv
          lo, hi = plsc.unpack(p, format=plsc.PackFormat.INTERLEAVED)
          buf[b, h, pl.ds(u * 32, _LANES)] = lo
          buf[b, h, pl.ds(u * 32 + _LANES, _LANES)] = hi

  row_base = base // 50

  def start_copy(c, buf, sem):
    cp = pltpu.make_async_copy(
        buf, out_hbm3.at[pl.ds(row_base + c * rows_per_chunk, rows_per_chunk)],
        sem)
    cp.start()

  def drain(buf, sem):
    pltpu.make_async_copy(
        buf, out_hbm3.at[pl.ds(row_base, rows_per_chunk)], sem).wait()

  compute(0, buf0)
  start_copy(0, buf0, sem0)
  compute(1, buf1)
  start_copy(1, buf1, sem1)

  def pair(p, carry):
    c0 = 2 * p
    drain(buf0, sem0)
    compute(c0, buf0)
    start_copy(c0, buf0, sem0)
    drain(buf1, sem1)
    compute(c0 + 1, buf1)
    start_copy(c0 + 1, buf1, sem1)
    return carry

  lax.fori_loop(1, nchunk // 2, pair, 0)
  drain(buf0, sem0)
  drain(buf1, sem1)


def kernel(inputs, q_table, r_table):
  total = inputs.shape[0] * inputs.shape[1]
  per_w = total // _NW
  chunk = 400
  idx = jnp.reshape(inputs, (total,)).astype(jnp.int32)

  def _prep(t):
    t = t.astype(jnp.bfloat16)
    t = t.reshape(_BUCKETS, _EMBED // 32, 2, _LANES)
    t = t.transpose(0, 1, 3, 2)
    return t.reshape(_BUCKETS * _EMBED)

  qt = _prep(q_table)
  rt = _prep(r_table)

  mesh = plsc.VectorSubcoreMesh(core_axis_name="c", subcore_axis_name="s")
  body = functools.partial(_qr_body, total, per_w, chunk)
  out = pl.kernel(
      body,
      out_type=jax.ShapeDtypeStruct(
          (inputs.shape[0], inputs.shape[1], _EMBED), jnp.float32),
      mesh=mesh,
      compiler_params=pltpu.CompilerParams(
          needs_layout_passes=False, use_tc_tiling_on_sc=False),
      scratch_types=[
          pltpu.VMEM((_BUCKETS * _EMBED,), jnp.bfloat16),
          pltpu.VMEM((_BUCKETS * _EMBED,), jnp.bfloat16),
          pltpu.VMEM((per_w,), jnp.int32),
          pltpu.VMEM((chunk // 50, 50, _EMBED), jnp.float32),
          pltpu.VMEM((chunk // 50, 50, _EMBED), jnp.float32),
          pltpu.SemaphoreType.DMA,
          pltpu.SemaphoreType.DMA,
      ],
  )(idx, qt, rt)
  return out


# speedup vs baseline: 5.9816x; 4.2014x over previous
import functools

import jax
import jax.numpy as jnp
from jax import lax
from jax.experimental import pallas as pl
from jax.experimental.pallas import tpu as pltpu
from jax.experimental.pallas import tpu_sc as plsc

_BUCKETS = 320
_EMBED = 64
_NC = 2
_NS = 16
_NW = _NC * _NS
_LANES = 16
_PAIRS = _EMBED // 2
_STRIDE = _PAIRS + 1


def _qr_body(hist, hchunk, idx_hbm, qt_hbm, rt_hbm, out_hbm,
             qt_v, rt_v, idx_v, buf0, buf1, sem0, sem1):
  nchunk = hist // hchunk
  wid = lax.axis_index("s") * _NC + lax.axis_index("c")

  pltpu.sync_copy(qt_hbm, qt_v)
  pltpu.sync_copy(rt_hbm, rt_v)
  pltpu.sync_copy(idx_hbm.at[:, pl.ds(wid * 128, 128)], idx_v)

  def compute(c, buf):
    @plsc.parallel_loop(0, hchunk * 8)
    def group(hb):
      h_ = lax.shift_right_logical(hb, 3)
      bg = lax.bitwise_and(hb, 7) * _LANES
      iv = idx_v[c * hchunk + h_, pl.ds(bg, _LANES)]
      q = lax.shift_right_logical(lax.shift_right_logical(iv, 6) * 6554, 15)
      r = iv - q * _BUCKETS
      qb = q * _STRIDE
      rb = r * _STRIDE
      for p in range(_PAIRS):
        qi = plsc.load_gather(qt_v, [qb + p])
        ri = plsc.load_gather(rt_v, [rb + p])
        pr = plsc.bitcast(qi, jnp.bfloat16) * plsc.bitcast(ri, jnp.bfloat16)
        lo, hi = plsc.unpack(pr, format=plsc.PackFormat.INTERLEAVED)
        d = 2 * p
        buf[h_, d // 8, d % 8, pl.ds(bg, _LANES)] = lo
        buf[h_, (d + 1) // 8, (d + 1) % 8, pl.ds(bg, _LANES)] = hi

  def start_copy(c, buf, sem):
    pltpu.make_async_copy(
        buf, out_hbm.at[pl.ds(c * hchunk, hchunk), :, wid], sem).start()

  def drain(buf, sem):
    pltpu.make_async_copy(
        buf, out_hbm.at[pl.ds(0, hchunk), :, wid], sem).wait()

  compute(0, buf0)
  start_copy(0, buf0, sem0)
  compute(1, buf1)
  start_copy(1, buf1, sem1)

  def pair(p, carry):
    c0 = 2 * p
    drain(buf0, sem0)
    compute(c0, buf0)
    start_copy(c0, buf0, sem0)
    drain(buf1, sem1)
    compute(c0 + 1, buf1)
    start_copy(c0 + 1, buf1, sem1)
    return carry

  lax.fori_loop(1, nchunk // 2, pair, 0)
  drain(buf0, sem0)
  drain(buf1, sem1)


def _prep_table(t):
  t = t.astype(jnp.bfloat16).reshape(_BUCKETS, _PAIRS, 2)
  packed = lax.bitcast_convert_type(t, jnp.int32)
  return jnp.pad(packed, ((0, 0), (0, 1))).reshape(_BUCKETS * _STRIDE)


def kernel(inputs, q_table, r_table):
  batch, hist = inputs.shape
  hchunk = 5
  idx = jnp.transpose(inputs).astype(jnp.int32)
  qt = _prep_table(q_table)
  rt = _prep_table(r_table)

  mesh = plsc.VectorSubcoreMesh(core_axis_name="c", subcore_axis_name="s")
  body = functools.partial(_qr_body, hist, hchunk)
  out = pl.kernel(
      body,
      out_type=jax.ShapeDtypeStruct(
          (hist, _EMBED // 8, batch // 128, 8, 128), jnp.float32),
      mesh=mesh,
      compiler_params=pltpu.CompilerParams(
          needs_layout_passes=False, use_tc_tiling_on_sc=False),
      scratch_types=[
          pltpu.VMEM((_BUCKETS * _STRIDE,), jnp.int32),
          pltpu.VMEM((_BUCKETS * _STRIDE,), jnp.int32),
          pltpu.VMEM((hist, 128), jnp.int32),
          pltpu.VMEM((hchunk, _EMBED // 8, 8, 128), jnp.float32),
          pltpu.VMEM((hchunk, _EMBED // 8, 8, 128), jnp.float32),
          pltpu.SemaphoreType.DMA,
          pltpu.SemaphoreType.DMA,
      ],
  )(idx, qt, rt)
  return out.transpose(2, 4, 0, 1, 3).reshape(batch, hist, _EMBED)
